# Initial kernel scaffold; baseline (speedup 1.0000x reference)
import jax
import jax.numpy as jnp
from jax.experimental import pallas as pl


def kernel(x, edge_index, batch, W_in, b_in, W_hid, b_hid, W_out, b_out):
    raise NotImplementedError("write your pallas kernel here")


# speedup vs baseline: 11.3956x; 11.3956x over previous
import functools

import jax
import jax.numpy as jnp
from jax import lax
from jax.experimental import pallas as pl
from jax.experimental.pallas import tpu as pltpu, tpu_sc as plsc

N = 10000
E = 320000
F_IN = 128
OUT = 5
G = 16
Q = 13

NTILE = 16
NP = 10240
RPT = NP // NTILE
EPT32 = E // 32
CHUNK = 128
NCHUNK = -(-(E // NTILE) // CHUNK)
EPT16 = NCHUNK * CHUNK
EPAD = NTILE * EPT16

_mesh = plsc.VectorSubcoreMesh(core_axis_name="c", subcore_axis_name="s")
_sc_params = pltpu.CompilerParams(needs_layout_passes=False,
                                  use_tc_tiling_on_sc=False)


@functools.partial(
    pl.kernel,
    mesh=_mesh,
    out_type=jax.ShapeDtypeStruct((32, NP), jnp.float32),
    compiler_params=_sc_params,
    scratch_types=[
        pltpu.VMEM((EPT32,), jnp.int32),
        pltpu.VMEM((NP,), jnp.float32),
    ],
)
def _deg_kernel(col_hbm, out_hbm, colv, degv):
    cid = lax.axis_index("c")
    sid = lax.axis_index("s")
    wid = cid * NTILE + sid

    def zero_body(i, carry):
        degv[pl.ds(i * 16, 16)] = jnp.zeros((16,), jnp.float32)
        return carry

    lax.fori_loop(0, NP // 16, zero_body, 0)

    pltpu.sync_copy(col_hbm.at[pl.ds(wid * EPT32, EPT32)], colv)
    ones = jnp.full((16,), 1.0, jnp.float32)

    def add_body(j, carry):
        idx = colv[pl.ds(j * 16, 16)]
        plsc.addupdate_scatter(degv, [idx], ones)
        return carry

    lax.fori_loop(0, EPT32 // 16, add_body, 0)

    pltpu.sync_copy(degv, out_hbm.at[wid])


@functools.partial(
    pl.kernel,
    mesh=_mesh,
    out_type=(
        jax.ShapeDtypeStruct((Q, NP, G), jnp.float32),
        jax.ShapeDtypeStruct((NP, G), jnp.float32),
    ),
    compiler_params=_sc_params,
    scratch_types=[
        pltpu.VMEM((NCHUNK, CHUNK), jnp.int32),
        pltpu.VMEM((NCHUNK, CHUNK), jnp.int32),
        pltpu.VMEM((CHUNK, G), jnp.float32),
        pltpu.VMEM((RPT, G), jnp.float32),
        pltpu.VMEM((RPT, G), jnp.float32),
        pltpu.VMEM((RPT, G), jnp.float32),
        pltpu.VMEM((RPT, G), jnp.float32),
        pltpu.VMEM((RPT, G), jnp.float32),
        pltpu.VMEM((RPT, G), jnp.float32),
        pltpu.VMEM_SHARED((NP, G), jnp.float32),
    ],
)
def _prop_kernel(rows_hbm, cols_hbm, p0_hbm, v0_hbm, invdeg_hbm, dis_hbm,
                 vout_hbm, p_hbm, rowv, colv, gbuf, tbuf, vbuf, pbuf,
                 zbuf, invd, disv, t_sh):
    cid = lax.axis_index("c")
    sid = lax.axis_index("s")

    @pl.when(cid == 0)
    def _body():
        nsl = pl.ds(sid * RPT, RPT)
        pltpu.sync_copy(rows_hbm.at[sid], rowv)
        pltpu.sync_copy(cols_hbm.at[sid], colv)
        pltpu.sync_copy(invdeg_hbm.at[nsl], invd)
        pltpu.sync_copy(dis_hbm.at[nsl], disv)
        pltpu.sync_copy(p0_hbm.at[nsl], pbuf)
        pltpu.sync_copy(pbuf, p_hbm.at[nsl])
        pltpu.sync_copy(v0_hbm.at[nsl], vbuf)
        pltpu.sync_copy(vbuf, vout_hbm.at[0, nsl])

        def zb(i, carry):
            zbuf[i, :] = jnp.zeros((G,), jnp.float32)
            return carry

        lax.fori_loop(0, RPT, zb, 0)
        pltpu.sync_copy(zbuf, t_sh.at[nsl])
        plsc.subcore_barrier()

        def step(q, carry):
            def chunk(j, inner):
                pltpu.sync_copy(p_hbm.at[colv.at[j]], gbuf)
                pltpu.sync_copy(gbuf, t_sh.at[rowv.at[j]], add=True)
                return inner

            lax.fori_loop(0, NCHUNK, chunk, 0)
            plsc.subcore_barrier()

            pltpu.sync_copy(t_sh.at[nsl], tbuf)

            def scale(i, inner):
                t = tbuf[i, :]
                vbuf[i, :] = t * disv[i, :]
                pbuf[i, :] = t * invd[i, :]
                return inner

            lax.fori_loop(0, RPT, scale, 0)
            pltpu.sync_copy(vbuf, vout_hbm.at[q, nsl])
            pltpu.sync_copy(pbuf, p_hbm.at[nsl])
            pltpu.sync_copy(zbuf, t_sh.at[nsl])
            plsc.subcore_barrier()
            return carry

        lax.fori_loop(1, Q, step, 0)


BLK = 2048
NBLK = NP // BLK


def _contract_body(v_ref, x_ref, d_ref, br_ref, bo_ref, out_ref, yacc, uacc):
    pid = pl.program_id(0)

    @pl.when(pid == 0)
    def _init():
        yacc[...] = jnp.zeros_like(yacc)
        uacc[...] = jnp.zeros_like(uacc)

    vblk = v_ref[...]
    xblk = x_ref[...]
    uacc[...] += jnp.sum(vblk, axis=1)
    for q in range(Q):
        yq = lax.dot_general(vblk[q], xblk, (((0,), (0,)), ((), ())),
                             preferred_element_type=jnp.float32)
        yacc[q] += yq

    @pl.when(pid == NBLK - 1)
    def _fin():
        y = yacc[...]
        d = d_ref[...]
        acc = jnp.zeros((G, OUT), jnp.float32)
        for q in range(Q):
            acc = acc + lax.dot_general(y[q], d[q], (((1,), (0,)), ((), ())),
                                        preferred_element_type=jnp.float32)
        acc = acc + lax.dot_general(uacc[...], br_ref[...],
                                    (((0,), (0,)), ((), ())),
                                    preferred_element_type=jnp.float32)
        out_ref[...] = acc + bo_ref[...]


_contract = pl.pallas_call(
    _contract_body,
    grid=(NBLK,),
    in_specs=[
        pl.BlockSpec((Q, BLK, G), lambda i: (0, i, 0)),
        pl.BlockSpec((BLK, F_IN), lambda i: (i, 0)),
        pl.BlockSpec((Q, F_IN, OUT), lambda i: (0, 0, 0)),
        pl.BlockSpec((Q, OUT), lambda i: (0, 0)),
        pl.BlockSpec((1, OUT), lambda i: (0, 0)),
    ],
    out_specs=pl.BlockSpec((G, OUT), lambda i: (0, 0)),
    out_shape=jax.ShapeDtypeStruct((G, OUT), jnp.float32),
    scratch_shapes=[
        pltpu.VMEM((Q, G, F_IN), jnp.float32),
        pltpu.VMEM((Q, G), jnp.float32),
    ],
)


def _poly_conv(Wl, S):
    a, b = Wl.shape[0], S.shape[0]
    out = [None] * (a + b - 1)
    for i in range(a):
        for j in range(b):
            t = Wl[i] @ S[j]
            out[i + j] = t if out[i + j] is None else out[i + j] + t
    return jnp.stack(out)


def kernel(x, edge_index, batch, W_in, b_in, W_hid, b_hid, W_out, b_out):
    f32 = jnp.float32
    row = edge_index[0].astype(jnp.int32)
    col = edge_index[1].astype(jnp.int32)

    deg = jnp.sum(_deg_kernel(col), axis=0)
    dis = jnp.where(deg > 0, lax.rsqrt(jnp.maximum(deg, 1e-12)),
                    0.0).astype(f32)
    invdeg = dis * dis

    bounds = jnp.searchsorted(batch, jnp.arange(G + 1, dtype=batch.dtype))
    cnt = (bounds[1:] - bounds[:-1]).astype(f32)
    recip = 1.0 / jnp.maximum(cnt, 1.0)
    onehot = (batch[:, None] == jnp.arange(G, dtype=batch.dtype)[None, :])
    v0 = jnp.zeros((NP, G), f32).at[:N].set(onehot * recip[None, :])
    p0 = dis[:, None] * v0
    invdeg16 = jnp.broadcast_to(invdeg[:, None], (NP, G))
    dis16 = jnp.broadcast_to(dis[:, None], (NP, G))

    pad = jnp.full((EPAD - E,), NP - 1, jnp.int32)
    rowp = jnp.concatenate([row, pad]).reshape(NTILE, NCHUNK, CHUNK)
    colp = jnp.concatenate([col, pad]).reshape(NTILE, NCHUNK, CHUNK)

    vout, _ = _prop_kernel(rowp, colp, p0, v0, invdeg16, dis16)

    s4 = _poly_conv(W_hid[2], jnp.broadcast_to(W_out[None], (1,) + W_out.shape))
    s3 = _poly_conv(W_hid[1], s4)
    s2 = _poly_conv(W_hid[0], s3)
    d = _poly_conv(W_in, s2)
    br = jnp.zeros((Q, OUT), f32)
    br = br.at[:10].add(jnp.einsum("i,qio->qo", b_in, s2))
    br = br.at[:7].add(jnp.einsum("i,qio->qo", b_hid[0], s3))
    br = br.at[:4].add(jnp.einsum("i,qio->qo", b_hid[1], s4))
    br = br.at[0].add(b_hid[2] @ W_out)

    xp = jnp.zeros((NP, F_IN), f32).at[:N].set(x)
    return _contract(vout, xp, d, br, b_out.reshape(1, OUT))


# speedup vs baseline: 23.7312x; 2.0825x over previous
import functools

import jax
import jax.numpy as jnp
from jax import lax
from jax.experimental import pallas as pl
from jax.experimental.pallas import tpu as pltpu, tpu_sc as plsc

N = 10000
E = 320000
F_IN = 128
OUT = 5
G = 16
Q = 13

NTILE = 16
NP = 10240
RPT = NP // NTILE
EPT32 = E // 32
CHUNK = 128
NBUF = 8
NGROUP = -(-(E // NTILE) // (CHUNK * NBUF))
NCHUNK = NGROUP * NBUF
EPT16 = NCHUNK * CHUNK
EPAD = NTILE * EPT16

_mesh = plsc.VectorSubcoreMesh(core_axis_name="c", subcore_axis_name="s")
_sc_params = pltpu.CompilerParams(needs_layout_passes=False,
                                  use_tc_tiling_on_sc=False)


@functools.partial(
    pl.kernel,
    mesh=_mesh,
    out_type=jax.ShapeDtypeStruct((32, NP), jnp.float32),
    compiler_params=_sc_params,
    scratch_types=[
        pltpu.VMEM((EPT32,), jnp.int32),
        pltpu.VMEM((NP,), jnp.float32),
    ],
)
def _deg_kernel(col_hbm, out_hbm, colv, degv):
    cid = lax.axis_index("c")
    sid = lax.axis_index("s")
    wid = cid * NTILE + sid

    def zero_body(i, carry):
        degv[pl.ds(i * 16, 16)] = jnp.zeros((16,), jnp.float32)
        return carry

    lax.fori_loop(0, NP // 16, zero_body, 0)

    pltpu.sync_copy(col_hbm.at[pl.ds(wid * EPT32, EPT32)], colv)
    ones = jnp.full((16,), 1.0, jnp.float32)

    def add_body(j, carry):
        idx = colv[pl.ds(j * 16, 16)]
        plsc.addupdate_scatter(degv, [idx], ones)
        return carry

    lax.fori_loop(0, EPT32 // 16, add_body, 0)

    pltpu.sync_copy(degv, out_hbm.at[wid])


@functools.partial(
    pl.kernel,
    mesh=_mesh,
    out_type=(
        jax.ShapeDtypeStruct((Q, NP, G), jnp.float32),
        jax.ShapeDtypeStruct((NP, G), jnp.float32),
    ),
    compiler_params=_sc_params,
    scratch_types=[
        pltpu.VMEM((NCHUNK, CHUNK), jnp.int32),
        pltpu.VMEM((NCHUNK, CHUNK), jnp.int32),
        [pltpu.VMEM((CHUNK, G), jnp.float32) for _ in range(NBUF)],
        [pltpu.SemaphoreType.DMA for _ in range(NBUF)],
        [pltpu.SemaphoreType.DMA for _ in range(NBUF)],
        pltpu.VMEM((RPT, G), jnp.float32),
        pltpu.VMEM((RPT, G), jnp.float32),
        pltpu.VMEM((RPT, G), jnp.float32),
        pltpu.VMEM((RPT, G), jnp.float32),
        pltpu.VMEM((RPT, G), jnp.float32),
        pltpu.VMEM((RPT, G), jnp.float32),
        pltpu.VMEM_SHARED((NP, G), jnp.float32),
    ],
)
def _prop_kernel(rows_hbm, cols_hbm, p0_hbm, v0_hbm, invdeg_hbm, dis_hbm,
                 vout_hbm, p_hbm, rowv, colv, gb, sg, ss, tbuf, vbuf, pbuf,
                 zbuf, invd, disv, t_sh):
    cid = lax.axis_index("c")
    sid = lax.axis_index("s")

    @pl.when(cid == 0)
    def _body():
        nsl = pl.ds(sid * RPT, RPT)
        pltpu.sync_copy(rows_hbm.at[sid], rowv)
        pltpu.sync_copy(cols_hbm.at[sid], colv)
        pltpu.sync_copy(invdeg_hbm.at[nsl], invd)
        pltpu.sync_copy(dis_hbm.at[nsl], disv)
        pltpu.sync_copy(p0_hbm.at[nsl], pbuf)
        pltpu.sync_copy(pbuf, p_hbm.at[nsl])
        pltpu.sync_copy(v0_hbm.at[nsl], vbuf)
        pltpu.sync_copy(vbuf, vout_hbm.at[0, nsl])

        def zb(i, carry):
            zbuf[i, :] = jnp.zeros((G,), jnp.float32)
            return carry

        lax.fori_loop(0, RPT, zb, 0)
        pltpu.sync_copy(zbuf, t_sh.at[nsl])
        plsc.subcore_barrier()

        def step(q, carry):
            for b in range(NBUF):
                pltpu.async_copy(p_hbm.at[colv.at[b]], gb[b], sg[b])

            def grp(jo, inner):
                for b in range(NBUF):
                    j = jo * NBUF + b
                    pltpu.make_async_copy(
                        p_hbm.at[colv.at[j]], gb[b], sg[b]).wait()
                    pltpu.async_copy(gb[b], t_sh.at[rowv.at[j]], ss[b],
                                     add=True)
                for b in range(NBUF):
                    @pl.when(jo < NGROUP - 1)
                    def _reuse(b=b):
                        jn = (jo + 1) * NBUF + b
                        pltpu.make_async_copy(
                            gb[b], t_sh.at[rowv.at[0]], ss[b]).wait()
                        pltpu.async_copy(p_hbm.at[colv.at[jn]], gb[b], sg[b])
                return inner

            lax.fori_loop(0, NGROUP, grp, 0)
            for b in range(NBUF):
                pltpu.make_async_copy(gb[b], t_sh.at[rowv.at[0]], ss[b]).wait()
            plsc.subcore_barrier()

            pltpu.sync_copy(t_sh.at[nsl], tbuf)

            def scale(i, inner):
                t = tbuf[i, :]
                vbuf[i, :] = t * disv[i, :]
                pbuf[i, :] = t * invd[i, :]
                return inner

            lax.fori_loop(0, RPT, scale, 0)
            pltpu.sync_copy(vbuf, vout_hbm.at[q, nsl])
            pltpu.sync_copy(pbuf, p_hbm.at[nsl])
            pltpu.sync_copy(zbuf, t_sh.at[nsl])
            plsc.subcore_barrier()
            return carry

        lax.fori_loop(1, Q, step, 0)


BLK = 2048
NBLK = NP // BLK


def _contract_body(v_ref, x_ref, d_ref, br_ref, bo_ref, out_ref, yacc, uacc):
    pid = pl.program_id(0)

    @pl.when(pid == 0)
    def _init():
        yacc[...] = jnp.zeros_like(yacc)
        uacc[...] = jnp.zeros_like(uacc)

    vblk = v_ref[...]
    xblk = x_ref[...]
    uacc[...] += jnp.sum(vblk, axis=1)
    for q in range(Q):
        yq = lax.dot_general(vblk[q], xblk, (((0,), (0,)), ((), ())),
                             preferred_element_type=jnp.float32)
        yacc[q] += yq

    @pl.when(pid == NBLK - 1)
    def _fin():
        y = yacc[...]
        d = d_ref[...]
        acc = jnp.zeros((G, OUT), jnp.float32)
        for q in range(Q):
            acc = acc + lax.dot_general(y[q], d[q], (((1,), (0,)), ((), ())),
                                        preferred_element_type=jnp.float32)
        acc = acc + lax.dot_general(uacc[...], br_ref[...],
                                    (((0,), (0,)), ((), ())),
                                    preferred_element_type=jnp.float32)
        out_ref[...] = acc + bo_ref[...]


_contract = pl.pallas_call(
    _contract_body,
    grid=(NBLK,),
    in_specs=[
        pl.BlockSpec((Q, BLK, G), lambda i: (0, i, 0)),
        pl.BlockSpec((BLK, F_IN), lambda i: (i, 0)),
        pl.BlockSpec((Q, F_IN, OUT), lambda i: (0, 0, 0)),
        pl.BlockSpec((Q, OUT), lambda i: (0, 0)),
        pl.BlockSpec((1, OUT), lambda i: (0, 0)),
    ],
    out_specs=pl.BlockSpec((G, OUT), lambda i: (0, 0)),
    out_shape=jax.ShapeDtypeStruct((G, OUT), jnp.float32),
    scratch_shapes=[
        pltpu.VMEM((Q, G, F_IN), jnp.float32),
        pltpu.VMEM((Q, G), jnp.float32),
    ],
)


def _poly_conv(Wl, S):
    a, b = Wl.shape[0], S.shape[0]
    out = [None] * (a + b - 1)
    for i in range(a):
        for j in range(b):
            t = Wl[i] @ S[j]
            out[i + j] = t if out[i + j] is None else out[i + j] + t
    return jnp.stack(out)


def kernel(x, edge_index, batch, W_in, b_in, W_hid, b_hid, W_out, b_out):
    f32 = jnp.float32
    row = edge_index[0].astype(jnp.int32)
    col = edge_index[1].astype(jnp.int32)

    deg = jnp.sum(_deg_kernel(col), axis=0)
    dis = jnp.where(deg > 0, lax.rsqrt(jnp.maximum(deg, 1e-12)),
                    0.0).astype(f32)
    invdeg = dis * dis

    bounds = jnp.searchsorted(batch, jnp.arange(G + 1, dtype=batch.dtype))
    cnt = (bounds[1:] - bounds[:-1]).astype(f32)
    recip = 1.0 / jnp.maximum(cnt, 1.0)
    onehot = (batch[:, None] == jnp.arange(G, dtype=batch.dtype)[None, :])
    v0 = jnp.zeros((NP, G), f32).at[:N].set(onehot * recip[None, :])
    p0 = dis[:, None] * v0
    invdeg16 = jnp.broadcast_to(invdeg[:, None], (NP, G))
    dis16 = jnp.broadcast_to(dis[:, None], (NP, G))

    pad = jnp.full((EPAD - E,), NP - 1, jnp.int32)
    rowp = jnp.concatenate([row, pad]).reshape(NTILE, NCHUNK, CHUNK)
    colp = jnp.concatenate([col, pad]).reshape(NTILE, NCHUNK, CHUNK)

    vout, _ = _prop_kernel(rowp, colp, p0, v0, invdeg16, dis16)

    s4 = _poly_conv(W_hid[2], jnp.broadcast_to(W_out[None], (1,) + W_out.shape))
    s3 = _poly_conv(W_hid[1], s4)
    s2 = _poly_conv(W_hid[0], s3)
    d = _poly_conv(W_in, s2)
    br = jnp.zeros((Q, OUT), f32)
    br = br.at[:10].add(jnp.einsum("i,qio->qo", b_in, s2))
    br = br.at[:7].add(jnp.einsum("i,qio->qo", b_hid[0], s3))
    br = br.at[:4].add(jnp.einsum("i,qio->qo", b_hid[1], s4))
    br = br.at[0].add(b_hid[2] @ W_out)

    xp = jnp.zeros((NP, F_IN), f32).at[:N].set(x)
    return _contract(vout, xp, d, br, b_out.reshape(1, OUT))


# speedup vs baseline: 30.3920x; 1.2807x over previous
import functools

import jax
import jax.numpy as jnp
from jax import lax
from jax.experimental import pallas as pl
from jax.experimental.pallas import tpu as pltpu, tpu_sc as plsc

N = 10000
E = 320000
F_IN = 128
OUT = 5
G = 16
Q = 13

NTILE = 16
NP = 10240
RPT = NP // NTILE
EPT32 = E // 32
CHUNK = 128
NBUF = 4
NGROUP = -(-(E // NTILE) // (CHUNK * NBUF))
NCHUNK = NGROUP * NBUF
EPT16 = NCHUNK * CHUNK
EPAD = NTILE * EPT16

_mesh = plsc.VectorSubcoreMesh(core_axis_name="c", subcore_axis_name="s")
_sc_params = pltpu.CompilerParams(needs_layout_passes=False,
                                  use_tc_tiling_on_sc=False)


@functools.partial(
    pl.kernel,
    mesh=_mesh,
    out_type=jax.ShapeDtypeStruct((32, NP), jnp.float32),
    compiler_params=_sc_params,
    scratch_types=[
        pltpu.VMEM((EPT32,), jnp.int32),
        pltpu.VMEM((NP,), jnp.float32),
    ],
)
def _deg_kernel(col_hbm, out_hbm, colv, degv):
    cid = lax.axis_index("c")
    sid = lax.axis_index("s")
    wid = cid * NTILE + sid

    def zero_body(i, carry):
        degv[pl.ds(i * 16, 16)] = jnp.zeros((16,), jnp.float32)
        return carry

    lax.fori_loop(0, NP // 16, zero_body, 0)

    pltpu.sync_copy(col_hbm.at[pl.ds(wid * EPT32, EPT32)], colv)
    ones = jnp.full((16,), 1.0, jnp.float32)

    def add_body(j, carry):
        idx = colv[pl.ds(j * 16, 16)]
        plsc.addupdate_scatter(degv, [idx], ones)
        return carry

    lax.fori_loop(0, EPT32 // 16, add_body, 0)

    pltpu.sync_copy(degv, out_hbm.at[wid])


@functools.partial(
    pl.kernel,
    mesh=_mesh,
    out_type=jax.ShapeDtypeStruct((Q, NP, G), jnp.float32),
    compiler_params=_sc_params,
    scratch_types=[
        pltpu.VMEM((NCHUNK, CHUNK), jnp.int32),
        pltpu.VMEM((NCHUNK, CHUNK), jnp.int32),
        [pltpu.VMEM((CHUNK, G), jnp.float32) for _ in range(NBUF)],
        [pltpu.SemaphoreType.DMA for _ in range(NBUF)],
        [pltpu.SemaphoreType.DMA for _ in range(NBUF)],
        pltpu.VMEM((RPT, G), jnp.float32),
        pltpu.VMEM((RPT, G), jnp.float32),
        pltpu.VMEM((RPT, G), jnp.float32),
        pltpu.VMEM((RPT, G), jnp.float32),
        pltpu.VMEM((RPT, G), jnp.float32),
        pltpu.VMEM((RPT, G), jnp.float32),
        pltpu.VMEM_SHARED((NP, G), jnp.float32),
        pltpu.VMEM_SHARED((NP, G), jnp.float32),
    ],
)
def _prop_kernel(rows_hbm, cols_hbm, p0_hbm, v0_hbm, invdeg_hbm, dis_hbm,
                 vout_hbm, rowv, colv, gb, sg, ss, tbuf, vbuf, pbuf,
                 zbuf, invd, disv, t_sh, p_sh):
    cid = lax.axis_index("c")
    sid = lax.axis_index("s")

    @pl.when(cid == 0)
    def _body():
        nsl = pl.ds(sid * RPT, RPT)
        pltpu.sync_copy(rows_hbm.at[sid], rowv)
        pltpu.sync_copy(cols_hbm.at[sid], colv)
        pltpu.sync_copy(invdeg_hbm.at[nsl], invd)
        pltpu.sync_copy(dis_hbm.at[nsl], disv)
        pltpu.sync_copy(p0_hbm.at[nsl], pbuf)
        pltpu.sync_copy(pbuf, p_sh.at[nsl])
        pltpu.sync_copy(v0_hbm.at[nsl], vbuf)
        pltpu.sync_copy(vbuf, vout_hbm.at[0, nsl])

        def zb(i, carry):
            zbuf[i, :] = jnp.zeros((G,), jnp.float32)
            return carry

        lax.fori_loop(0, RPT, zb, 0)
        pltpu.sync_copy(zbuf, t_sh.at[nsl])
        plsc.subcore_barrier()

        def step(q, carry):
            for b in range(NBUF):
                pltpu.async_copy(p_sh.at[colv.at[b]], gb[b], sg[b])

            def grp(jo, inner):
                for b in range(NBUF):
                    j = jo * NBUF + b
                    pltpu.make_async_copy(
                        p_sh.at[colv.at[j]], gb[b], sg[b]).wait()
                    pltpu.async_copy(gb[b], t_sh.at[rowv.at[j]], ss[b],
                                     add=True)
                for b in range(NBUF):
                    @pl.when(jo < NGROUP - 1)
                    def _reuse(b=b):
                        jn = (jo + 1) * NBUF + b
                        pltpu.make_async_copy(
                            gb[b], t_sh.at[rowv.at[0]], ss[b]).wait()
                        pltpu.async_copy(p_sh.at[colv.at[jn]], gb[b], sg[b])
                return inner

            lax.fori_loop(0, NGROUP, grp, 0)
            for b in range(NBUF):
                pltpu.make_async_copy(gb[b], t_sh.at[rowv.at[0]], ss[b]).wait()
            plsc.subcore_barrier()

            pltpu.sync_copy(t_sh.at[nsl], tbuf)

            def scale(i, inner):
                t = tbuf[i, :]
                vbuf[i, :] = t * disv[i, :]
                pbuf[i, :] = t * invd[i, :]
                return inner

            lax.fori_loop(0, RPT, scale, 0)
            pltpu.sync_copy(vbuf, vout_hbm.at[q, nsl])
            pltpu.sync_copy(pbuf, p_sh.at[nsl])
            pltpu.sync_copy(zbuf, t_sh.at[nsl])
            plsc.subcore_barrier()
            return carry

        lax.fori_loop(1, Q, step, 0)


BLK = 2048
NBLK = NP // BLK


def _contract_body(v_ref, x_ref, d_ref, br_ref, bo_ref, out_ref, yacc, uacc):
    pid = pl.program_id(0)

    @pl.when(pid == 0)
    def _init():
        yacc[...] = jnp.zeros_like(yacc)
        uacc[...] = jnp.zeros_like(uacc)

    vblk = v_ref[...]
    xblk = x_ref[...]
    uacc[...] += jnp.sum(vblk, axis=1)
    for q in range(Q):
        yq = lax.dot_general(vblk[q], xblk, (((0,), (0,)), ((), ())),
                             preferred_element_type=jnp.float32)
        yacc[q] += yq

    @pl.when(pid == NBLK - 1)
    def _fin():
        y = yacc[...]
        d = d_ref[...]
        acc = jnp.zeros((G, OUT), jnp.float32)
        for q in range(Q):
            acc = acc + lax.dot_general(y[q], d[q], (((1,), (0,)), ((), ())),
                                        preferred_element_type=jnp.float32)
        acc = acc + lax.dot_general(uacc[...], br_ref[...],
                                    (((0,), (0,)), ((), ())),
                                    preferred_element_type=jnp.float32)
        out_ref[...] = acc + bo_ref[...]


_contract = pl.pallas_call(
    _contract_body,
    grid=(NBLK,),
    in_specs=[
        pl.BlockSpec((Q, BLK, G), lambda i: (0, i, 0)),
        pl.BlockSpec((BLK, F_IN), lambda i: (i, 0)),
        pl.BlockSpec((Q, F_IN, OUT), lambda i: (0, 0, 0)),
        pl.BlockSpec((Q, OUT), lambda i: (0, 0)),
        pl.BlockSpec((1, OUT), lambda i: (0, 0)),
    ],
    out_specs=pl.BlockSpec((G, OUT), lambda i: (0, 0)),
    out_shape=jax.ShapeDtypeStruct((G, OUT), jnp.float32),
    scratch_shapes=[
        pltpu.VMEM((Q, G, F_IN), jnp.float32),
        pltpu.VMEM((Q, G), jnp.float32),
    ],
)


def _poly_conv(Wl, S):
    a, b = Wl.shape[0], S.shape[0]
    out = [None] * (a + b - 1)
    for i in range(a):
        for j in range(b):
            t = Wl[i] @ S[j]
            out[i + j] = t if out[i + j] is None else out[i + j] + t
    return jnp.stack(out)


def kernel(x, edge_index, batch, W_in, b_in, W_hid, b_hid, W_out, b_out):
    f32 = jnp.float32
    row = edge_index[0].astype(jnp.int32)
    col = edge_index[1].astype(jnp.int32)

    deg = jnp.sum(_deg_kernel(col), axis=0)
    dis = jnp.where(deg > 0, lax.rsqrt(jnp.maximum(deg, 1e-12)),
                    0.0).astype(f32)
    invdeg = dis * dis

    bounds = jnp.searchsorted(batch, jnp.arange(G + 1, dtype=batch.dtype))
    cnt = (bounds[1:] - bounds[:-1]).astype(f32)
    recip = 1.0 / jnp.maximum(cnt, 1.0)
    onehot = (batch[:, None] == jnp.arange(G, dtype=batch.dtype)[None, :])
    v0 = jnp.zeros((NP, G), f32).at[:N].set(onehot * recip[None, :])
    p0 = dis[:, None] * v0
    invdeg16 = jnp.broadcast_to(invdeg[:, None], (NP, G))
    dis16 = jnp.broadcast_to(dis[:, None], (NP, G))

    pad = jnp.full((EPAD - E,), NP - 1, jnp.int32)
    rowp = jnp.concatenate([row, pad]).reshape(NTILE, NCHUNK, CHUNK)
    colp = jnp.concatenate([col, pad]).reshape(NTILE, NCHUNK, CHUNK)

    vout = _prop_kernel(rowp, colp, p0, v0, invdeg16, dis16)

    s4 = _poly_conv(W_hid[2], jnp.broadcast_to(W_out[None], (1,) + W_out.shape))
    s3 = _poly_conv(W_hid[1], s4)
    s2 = _poly_conv(W_hid[0], s3)
    d = _poly_conv(W_in, s2)
    br = jnp.zeros((Q, OUT), f32)
    br = br.at[:10].add(jnp.einsum("i,qio->qo", b_in, s2))
    br = br.at[:7].add(jnp.einsum("i,qio->qo", b_hid[0], s3))
    br = br.at[:4].add(jnp.einsum("i,qio->qo", b_hid[1], s4))
    br = br.at[0].add(b_hid[2] @ W_out)

    xp = jnp.zeros((NP, F_IN), f32).at[:N].set(x)
    return _contract(vout, xp, d, br, b_out.reshape(1, OUT))


# speedup vs baseline: 37.6603x; 1.2392x over previous
import functools

import jax
import jax.numpy as jnp
from jax import lax
from jax.experimental import pallas as pl
from jax.experimental.pallas import tpu as pltpu, tpu_sc as plsc

N = 10000
E = 320000
F_IN = 128
OUT = 5
G = 16
GH = 8
Q = 13

NTILE = 16
NP = 10240
RPT = NP // NTILE
HPT = RPT // 2
EPT32 = E // 32
CHUNK = 128
NBUF = 8
NGROUP = -(-(E // NTILE) // (CHUNK * NBUF))
NCHUNK = NGROUP * NBUF
EPT16 = NCHUNK * CHUNK
EPAD = NTILE * EPT16

_mesh = plsc.VectorSubcoreMesh(core_axis_name="c", subcore_axis_name="s")
_sc_params = pltpu.CompilerParams(needs_layout_passes=False,
                                  use_tc_tiling_on_sc=False)


@functools.partial(
    pl.kernel,
    mesh=_mesh,
    out_type=jax.ShapeDtypeStruct((32, NP), jnp.float32),
    compiler_params=_sc_params,
    scratch_types=[
        pltpu.VMEM((EPT32,), jnp.int32),
        pltpu.VMEM((NP,), jnp.float32),
    ],
)
def _deg_kernel(col_hbm, out_hbm, colv, degv):
    cid = lax.axis_index("c")
    sid = lax.axis_index("s")
    wid = cid * NTILE + sid

    def zero_body(i, carry):
        degv[pl.ds(i * 16, 16)] = jnp.zeros((16,), jnp.float32)
        return carry

    lax.fori_loop(0, NP // 16, zero_body, 0)

    pltpu.sync_copy(col_hbm.at[pl.ds(wid * EPT32, EPT32)], colv)
    ones = jnp.full((16,), 1.0, jnp.float32)

    def add_body(j, carry):
        idx = colv[pl.ds(j * 16, 16)]
        plsc.addupdate_scatter(degv, [idx], ones)
        return carry

    lax.fori_loop(0, EPT32 // 16, add_body, 0)

    pltpu.sync_copy(degv, out_hbm.at[wid])


@functools.partial(
    pl.kernel,
    mesh=_mesh,
    out_type=jax.ShapeDtypeStruct((2, Q, NP, GH), jnp.float32),
    compiler_params=_sc_params,
    scratch_types=[
        pltpu.VMEM((NCHUNK, CHUNK), jnp.int32),
        pltpu.VMEM((NCHUNK, CHUNK), jnp.int32),
        [pltpu.VMEM((CHUNK, GH), jnp.float32) for _ in range(NBUF)],
        [pltpu.SemaphoreType.DMA for _ in range(NBUF)],
        [pltpu.SemaphoreType.DMA for _ in range(NBUF)],
        pltpu.VMEM((RPT, GH), jnp.float32),
        pltpu.VMEM((RPT, GH), jnp.float32),
        pltpu.VMEM((RPT, GH), jnp.float32),
        pltpu.VMEM((RPT, GH), jnp.float32),
        pltpu.VMEM((HPT, G), jnp.float32),
        pltpu.VMEM((HPT, G), jnp.float32),
        pltpu.VMEM_SHARED((NP, GH), jnp.float32),
        pltpu.VMEM_SHARED((NP, GH), jnp.float32),
    ],
)
def _prop_kernel(rows_hbm, cols_hbm, p0_hbm, v0_hbm, invdeg_hbm, dis_hbm,
                 vout_hbm, rowv, colv, gb, sg, ss, tbuf, vbuf, pbuf,
                 zbuf, invd, disv, t_sh, p_sh):
    cid = lax.axis_index("c")
    sid = lax.axis_index("s")

    nsl = pl.ds(sid * RPT, RPT)
    hsl = pl.ds(sid * HPT, HPT)
    pltpu.sync_copy(rows_hbm.at[sid], rowv)
    pltpu.sync_copy(cols_hbm.at[sid], colv)
    pltpu.sync_copy(invdeg_hbm.at[hsl], invd)
    pltpu.sync_copy(dis_hbm.at[hsl], disv)
    pltpu.sync_copy(p0_hbm.at[cid, nsl], pbuf)
    pltpu.sync_copy(pbuf, p_sh.at[nsl])
    pltpu.sync_copy(v0_hbm.at[cid, nsl], vbuf)
    pltpu.sync_copy(vbuf, vout_hbm.at[cid, 0, nsl])

    lane = lax.iota(jnp.int32, 16)
    roff = lax.shift_right_logical(lane, 3)
    coff = lax.bitwise_and(lane, 7)
    zeros16 = jnp.zeros((G,), jnp.float32)

    def zb(i, carry):
        plsc.store_scatter(zbuf, [2 * i + roff, coff], zeros16)
        return carry

    lax.fori_loop(0, HPT, zb, 0)
    pltpu.sync_copy(zbuf, t_sh.at[nsl])
    plsc.subcore_barrier()

    def step(q, carry):
        for b in range(NBUF):
            pltpu.async_copy(p_sh.at[colv.at[b]], gb[b], sg[b])

        def grp(jo, inner):
            for b in range(NBUF):
                j = jo * NBUF + b
                pltpu.make_async_copy(
                    p_sh.at[colv.at[j]], gb[b], sg[b]).wait()
                pltpu.async_copy(gb[b], t_sh.at[rowv.at[j]], ss[b],
                                 add=True)
            for b in range(NBUF):
                @pl.when(jo < NGROUP - 1)
                def _reuse(b=b):
                    jn = (jo + 1) * NBUF + b
                    pltpu.make_async_copy(
                        gb[b], t_sh.at[rowv.at[0]], ss[b]).wait()
                    pltpu.async_copy(p_sh.at[colv.at[jn]], gb[b], sg[b])
            return inner

        lax.fori_loop(0, NGROUP, grp, 0)
        for b in range(NBUF):
            pltpu.make_async_copy(gb[b], t_sh.at[rowv.at[0]], ss[b]).wait()
        plsc.subcore_barrier()

        pltpu.sync_copy(t_sh.at[nsl], tbuf)

        def scale(i, inner):
            ridx = 2 * i + roff
            t = plsc.load_gather(tbuf, [ridx, coff])
            plsc.store_scatter(vbuf, [ridx, coff], t * disv[i, :])
            plsc.store_scatter(pbuf, [ridx, coff], t * invd[i, :])
            return inner

        lax.fori_loop(0, HPT, scale, 0)
        pltpu.sync_copy(vbuf, vout_hbm.at[cid, q, nsl])
        pltpu.sync_copy(pbuf, p_sh.at[nsl])
        pltpu.sync_copy(zbuf, t_sh.at[nsl])
        plsc.subcore_barrier()
        return carry

    lax.fori_loop(1, Q, step, 0)


BLK = 2048
NBLK = NP // BLK


def _contract_body(v_ref, x_ref, d_ref, br_ref, bo_ref, out_ref, yacc, uacc):
    pid = pl.program_id(0)

    @pl.when(pid == 0)
    def _init():
        yacc[...] = jnp.zeros_like(yacc)
        uacc[...] = jnp.zeros_like(uacc)

    vblk = v_ref[...]
    xblk = x_ref[...]
    uacc[...] += jnp.sum(vblk, axis=2)
    for c in range(2):
        for q in range(Q):
            yq = lax.dot_general(vblk[c, q], xblk, (((0,), (0,)), ((), ())),
                                 preferred_element_type=jnp.float32)
            yacc[c, q] += yq

    @pl.when(pid == NBLK - 1)
    def _fin():
        y = jnp.concatenate([yacc[0], yacc[1]], axis=1)
        u = jnp.concatenate([uacc[0], uacc[1]], axis=1)
        d = d_ref[...]
        acc = jnp.zeros((G, OUT), jnp.float32)
        for q in range(Q):
            acc = acc + lax.dot_general(y[q], d[q], (((1,), (0,)), ((), ())),
                                        preferred_element_type=jnp.float32)
        acc = acc + lax.dot_general(u, br_ref[...],
                                    (((0,), (0,)), ((), ())),
                                    preferred_element_type=jnp.float32)
        out_ref[...] = acc + bo_ref[...]


_contract = pl.pallas_call(
    _contract_body,
    grid=(NBLK,),
    in_specs=[
        pl.BlockSpec((2, Q, BLK, GH), lambda i: (0, 0, i, 0)),
        pl.BlockSpec((BLK, F_IN), lambda i: (i, 0)),
        pl.BlockSpec((Q, F_IN, OUT), lambda i: (0, 0, 0)),
        pl.BlockSpec((Q, OUT), lambda i: (0, 0)),
        pl.BlockSpec((1, OUT), lambda i: (0, 0)),
    ],
    out_specs=pl.BlockSpec((G, OUT), lambda i: (0, 0)),
    out_shape=jax.ShapeDtypeStruct((G, OUT), jnp.float32),
    scratch_shapes=[
        pltpu.VMEM((2, Q, GH, F_IN), jnp.float32),
        pltpu.VMEM((2, Q, GH), jnp.float32),
    ],
)


def _poly_conv(Wl, S):
    a, b = Wl.shape[0], S.shape[0]
    out = [None] * (a + b - 1)
    for i in range(a):
        for j in range(b):
            t = Wl[i] @ S[j]
            out[i + j] = t if out[i + j] is None else out[i + j] + t
    return jnp.stack(out)


def kernel(x, edge_index, batch, W_in, b_in, W_hid, b_hid, W_out, b_out):
    f32 = jnp.float32
    row = edge_index[0].astype(jnp.int32)
    col = edge_index[1].astype(jnp.int32)

    deg = jnp.sum(_deg_kernel(col), axis=0)
    dis = jnp.where(deg > 0, lax.rsqrt(jnp.maximum(deg, 1e-12)),
                    0.0).astype(f32)
    invdeg = dis * dis

    bounds = jnp.searchsorted(batch, jnp.arange(G + 1, dtype=batch.dtype))
    cnt = (bounds[1:] - bounds[:-1]).astype(f32)
    recip = 1.0 / jnp.maximum(cnt, 1.0)
    onehot = (batch[:, None] == jnp.arange(G, dtype=batch.dtype)[None, :])
    v0 = jnp.zeros((NP, G), f32).at[:N].set(onehot * recip[None, :])
    p0 = dis[:, None] * v0
    v0h = v0.reshape(NP, 2, GH).transpose(1, 0, 2)
    p0h = p0.reshape(NP, 2, GH).transpose(1, 0, 2)
    invdegP = jnp.broadcast_to(invdeg[:, None], (NP, GH)).reshape(NP // 2, G)
    disP = jnp.broadcast_to(dis[:, None], (NP, GH)).reshape(NP // 2, G)

    pad = jnp.full((EPAD - E,), NP - 1, jnp.int32)
    rowp = jnp.concatenate([row, pad]).reshape(NTILE, NCHUNK, CHUNK)
    colp = jnp.concatenate([col, pad]).reshape(NTILE, NCHUNK, CHUNK)

    vout = _prop_kernel(rowp, colp, p0h, v0h, invdegP, disP)

    s4 = _poly_conv(W_hid[2], jnp.broadcast_to(W_out[None], (1,) + W_out.shape))
    s3 = _poly_conv(W_hid[1], s4)
    s2 = _poly_conv(W_hid[0], s3)
    d = _poly_conv(W_in, s2)
    br = jnp.zeros((Q, OUT), f32)
    br = br.at[:10].add(jnp.einsum("i,qio->qo", b_in, s2))
    br = br.at[:7].add(jnp.einsum("i,qio->qo", b_hid[0], s3))
    br = br.at[:4].add(jnp.einsum("i,qio->qo", b_hid[1], s4))
    br = br.at[0].add(b_hid[2] @ W_out)

    xp = jnp.zeros((NP, F_IN), f32).at[:N].set(x)
    return _contract(vout, xp, d, br, b_out.reshape(1, OUT))


# speedup vs baseline: 37.7576x; 1.0026x over previous
import functools

import jax
import jax.numpy as jnp
from jax import lax
from jax.experimental import pallas as pl
from jax.experimental.pallas import tpu as pltpu, tpu_sc as plsc

N = 10000
E = 320000
F_IN = 128
OUT = 5
G = 16
GH = 8
Q = 13

NTILE = 16
NP = 10240
RPT = NP // NTILE
HPT = RPT // 2
EPT32 = E // 32
EPT = E // NTILE
CHUNK = 128
NBUF = 8
NGROUP = -(-(E // NTILE) // (CHUNK * NBUF))
NCHUNK = NGROUP * NBUF
EPT16 = NCHUNK * CHUNK
EPAD = NTILE * EPT16

_mesh = plsc.VectorSubcoreMesh(core_axis_name="c", subcore_axis_name="s")
_sc_params = pltpu.CompilerParams(needs_layout_passes=False,
                                  use_tc_tiling_on_sc=False)


@functools.partial(
    pl.kernel,
    mesh=_mesh,
    out_type=jax.ShapeDtypeStruct((32, NP), jnp.float32),
    compiler_params=_sc_params,
    scratch_types=[
        pltpu.VMEM((EPT32,), jnp.int32),
        pltpu.VMEM((NP,), jnp.float32),
    ],
)
def _deg_kernel(col_hbm, out_hbm, colv, degv):
    cid = lax.axis_index("c")
    sid = lax.axis_index("s")
    wid = cid * NTILE + sid

    def zero_body(i, carry):
        degv[pl.ds(i * 16, 16)] = jnp.zeros((16,), jnp.float32)
        return carry

    lax.fori_loop(0, NP // 16, zero_body, 0)

    pltpu.sync_copy(col_hbm.at[pl.ds(wid * EPT32, EPT32)], colv)
    ones = jnp.full((16,), 1.0, jnp.float32)

    def add_body(j, carry):
        idx = colv[pl.ds(j * 16, 16)]
        plsc.addupdate_scatter(degv, [idx], ones)
        return carry

    lax.fori_loop(0, EPT32 // 16, add_body, 0)

    pltpu.sync_copy(degv, out_hbm.at[wid])


@functools.partial(
    pl.kernel,
    mesh=_mesh,
    out_type=jax.ShapeDtypeStruct((2, Q, NP, GH), jnp.float32),
    compiler_params=_sc_params,
    scratch_types=[
        pltpu.VMEM((EPT16,), jnp.int32),
        pltpu.VMEM((EPT16,), jnp.int32),
        [pltpu.VMEM((CHUNK, GH), jnp.float32) for _ in range(NBUF)],
        [pltpu.SemaphoreType.DMA for _ in range(NBUF)],
        [pltpu.SemaphoreType.DMA for _ in range(NBUF)],
        pltpu.VMEM((RPT, GH), jnp.float32),
        pltpu.VMEM((RPT, GH), jnp.float32),
        pltpu.VMEM((RPT, GH), jnp.float32),
        pltpu.VMEM((RPT, GH), jnp.float32),
        pltpu.VMEM((HPT, G), jnp.float32),
        pltpu.VMEM((HPT, G), jnp.float32),
        pltpu.VMEM((RPT,), jnp.int32),
        pltpu.VMEM((G,), jnp.float32),
        pltpu.VMEM_SHARED((NP, GH), jnp.float32),
        pltpu.VMEM_SHARED((NP, GH), jnp.float32),
    ],
)
def _prop_kernel(rows_hbm, cols_hbm, batch_hbm, recip_hbm, invdeg_hbm,
                 dis_hbm, vout_hbm, rowv, colv, gb, sg, ss, tbuf, vbuf, pbuf,
                 zbuf, invd, disv, batchv, recipv, t_sh, p_sh):
    cid = lax.axis_index("c")
    sid = lax.axis_index("s")

    nsl = pl.ds(sid * RPT, RPT)
    hsl = pl.ds(sid * HPT, HPT)
    pltpu.sync_copy(rows_hbm.at[pl.ds(sid * EPT, EPT)],
                    rowv.at[pl.ds(0, EPT)])
    pltpu.sync_copy(cols_hbm.at[pl.ds(sid * EPT, EPT)],
                    colv.at[pl.ds(0, EPT)])
    pltpu.sync_copy(invdeg_hbm.at[hsl], invd)
    pltpu.sync_copy(dis_hbm.at[hsl], disv)
    pltpu.sync_copy(batch_hbm.at[nsl], batchv)
    pltpu.sync_copy(recip_hbm, recipv)

    padidx = jnp.full((16,), NP - 1, jnp.int32)

    def padb(i, carry):
        rowv[pl.ds(EPT + 16 * i, 16)] = padidx
        colv[pl.ds(EPT + 16 * i, 16)] = padidx
        return carry

    lax.fori_loop(0, (EPT16 - EPT) // 16, padb, 0)

    lane = lax.iota(jnp.int32, 16)
    roff = lax.shift_right_logical(lane, 3)
    coff = lax.bitwise_and(lane, 7)
    zeros16 = jnp.zeros((G,), jnp.float32)

    def zb(i, carry):
        idx = [2 * i + roff, coff]
        plsc.store_scatter(zbuf, idx, zeros16)
        plsc.store_scatter(vbuf, idx, zeros16)
        plsc.store_scatter(pbuf, idx, zeros16)
        return carry

    lax.fori_loop(0, HPT, zb, 0)

    def vb(i, carry):
        b16 = batchv[pl.ds(16 * i, 16)]
        lane_g = b16 - 8 * cid
        mask = jnp.logical_and(lane_g >= 0, lane_g < GH)
        rv = plsc.load_gather(recipv, [jnp.minimum(b16, G - 1)])
        d16 = plsc.load_gather(
            disv, [8 * i + lax.shift_right_logical(lane, 1),
                   lax.bitwise_and(lane, 1) * GH])
        idx = [16 * i + lane, lane_g]
        plsc.store_scatter(vbuf, idx, rv, mask=mask)
        plsc.store_scatter(pbuf, idx, rv * d16, mask=mask)
        return carry

    lax.fori_loop(0, RPT // 16, vb, 0)

    pltpu.sync_copy(pbuf, p_sh.at[nsl])
    pltpu.sync_copy(vbuf, vout_hbm.at[cid, 0, nsl])
    pltpu.sync_copy(zbuf, t_sh.at[nsl])
    plsc.subcore_barrier()

    def step(q, carry):
        def cs(j):
            return pl.ds(j * CHUNK, CHUNK)

        for b in range(NBUF):
            pltpu.async_copy(p_sh.at[colv.at[cs(b)]], gb[b], sg[b])

        def grp(jo, inner):
            for b in range(NBUF):
                j = jo * NBUF + b
                pltpu.make_async_copy(
                    p_sh.at[colv.at[cs(j)]], gb[b], sg[b]).wait()
                pltpu.async_copy(gb[b], t_sh.at[rowv.at[cs(j)]], ss[b],
                                 add=True)
            for b in range(NBUF):
                @pl.when(jo < NGROUP - 1)
                def _reuse(b=b):
                    jn = (jo + 1) * NBUF + b
                    pltpu.make_async_copy(
                        gb[b], t_sh.at[rowv.at[cs(0)]], ss[b]).wait()
                    pltpu.async_copy(p_sh.at[colv.at[cs(jn)]], gb[b], sg[b])
            return inner

        lax.fori_loop(0, NGROUP, grp, 0)
        for b in range(NBUF):
            pltpu.make_async_copy(gb[b], t_sh.at[rowv.at[cs(0)]],
                                  ss[b]).wait()
        plsc.subcore_barrier()

        pltpu.sync_copy(t_sh.at[nsl], tbuf)

        def scale(i, inner):
            ridx = 2 * i + roff
            t = plsc.load_gather(tbuf, [ridx, coff])
            plsc.store_scatter(vbuf, [ridx, coff], t * disv[i, :])
            plsc.store_scatter(pbuf, [ridx, coff], t * invd[i, :])
            return inner

        lax.fori_loop(0, HPT, scale, 0)
        pltpu.sync_copy(vbuf, vout_hbm.at[cid, q, nsl])
        pltpu.sync_copy(pbuf, p_sh.at[nsl])
        pltpu.sync_copy(zbuf, t_sh.at[nsl])
        plsc.subcore_barrier()
        return carry

    lax.fori_loop(1, Q, step, 0)


BLK = 2048
NBLK = NP // BLK


def _contract_body(v_ref, x_ref, d_ref, br_ref, bo_ref, out_ref, yacc, uacc):
    pid = pl.program_id(0)

    @pl.when(pid == 0)
    def _init():
        yacc[...] = jnp.zeros_like(yacc)
        uacc[...] = jnp.zeros_like(uacc)

    vblk = v_ref[...]
    xblk = x_ref[...]
    uacc[...] += jnp.sum(vblk, axis=2)
    for c in range(2):
        for q in range(Q):
            yq = lax.dot_general(vblk[c, q], xblk, (((0,), (0,)), ((), ())),
                                 preferred_element_type=jnp.float32)
            yacc[c, q] += yq

    @pl.when(pid == NBLK - 1)
    def _fin():
        y = jnp.concatenate([yacc[0], yacc[1]], axis=1)
        u = jnp.concatenate([uacc[0], uacc[1]], axis=1)
        d = d_ref[...]
        acc = jnp.zeros((G, OUT), jnp.float32)
        for q in range(Q):
            acc = acc + lax.dot_general(y[q], d[q], (((1,), (0,)), ((), ())),
                                        preferred_element_type=jnp.float32)
        acc = acc + lax.dot_general(u, br_ref[...],
                                    (((0,), (0,)), ((), ())),
                                    preferred_element_type=jnp.float32)
        out_ref[...] = acc + bo_ref[...]


_contract = pl.pallas_call(
    _contract_body,
    grid=(NBLK,),
    in_specs=[
        pl.BlockSpec((2, Q, BLK, GH), lambda i: (0, 0, i, 0)),
        pl.BlockSpec((BLK, F_IN), lambda i: (i, 0)),
        pl.BlockSpec((Q, F_IN, OUT), lambda i: (0, 0, 0)),
        pl.BlockSpec((Q, OUT), lambda i: (0, 0)),
        pl.BlockSpec((1, OUT), lambda i: (0, 0)),
    ],
    out_specs=pl.BlockSpec((G, OUT), lambda i: (0, 0)),
    out_shape=jax.ShapeDtypeStruct((G, OUT), jnp.float32),
    scratch_shapes=[
        pltpu.VMEM((2, Q, GH, F_IN), jnp.float32),
        pltpu.VMEM((2, Q, GH), jnp.float32),
    ],
)


def _poly_conv(Wl, S):
    a, b = Wl.shape[0], S.shape[0]
    out = [None] * (a + b - 1)
    for i in range(a):
        for j in range(b):
            t = Wl[i] @ S[j]
            out[i + j] = t if out[i + j] is None else out[i + j] + t
    return jnp.stack(out)


def kernel(x, edge_index, batch, W_in, b_in, W_hid, b_hid, W_out, b_out):
    f32 = jnp.float32
    row = edge_index[0].astype(jnp.int32)
    col = edge_index[1].astype(jnp.int32)

    deg = jnp.sum(_deg_kernel(col), axis=0)
    dis = jnp.where(deg > 0, lax.rsqrt(jnp.maximum(deg, 1e-12)),
                    0.0).astype(f32)
    invdeg = dis * dis

    bounds = jnp.searchsorted(batch, jnp.arange(G + 1, dtype=batch.dtype))
    cnt = (bounds[1:] - bounds[:-1]).astype(f32)
    recip = 1.0 / jnp.maximum(cnt, 1.0)
    batchp = jnp.concatenate(
        [batch.astype(jnp.int32), jnp.full((NP - N,), G, jnp.int32)])
    invdegP = jnp.broadcast_to(invdeg[:, None], (NP, GH)).reshape(NP // 2, G)
    disP = jnp.broadcast_to(dis[:, None], (NP, GH)).reshape(NP // 2, G)

    vout = _prop_kernel(row, col, batchp, recip, invdegP, disP)

    s4 = _poly_conv(W_hid[2], jnp.broadcast_to(W_out[None], (1,) + W_out.shape))
    s3 = _poly_conv(W_hid[1], s4)
    s2 = _poly_conv(W_hid[0], s3)
    d = _poly_conv(W_in, s2)
    br = jnp.zeros((Q, OUT), f32)
    br = br.at[:10].add(jnp.einsum("i,qio->qo", b_in, s2))
    br = br.at[:7].add(jnp.einsum("i,qio->qo", b_hid[0], s3))
    br = br.at[:4].add(jnp.einsum("i,qio->qo", b_hid[1], s4))
    br = br.at[0].add(b_hid[2] @ W_out)

    xp = jnp.zeros((NP, F_IN), f32).at[:N].set(x)
    return _contract(vout, xp, d, br, b_out.reshape(1, OUT))
